# Initial kernel scaffold; baseline (speedup 1.0000x reference)
import jax
import jax.numpy as jnp
from jax.experimental import pallas as pl


def kernel(source_points, target_points, scale, translation):
    raise NotImplementedError("write your pallas kernel here")


# speedup vs baseline: 3.1374x; 3.1374x over previous
import functools

import jax
import jax.numpy as jnp
from jax import lax
from jax.experimental import pallas as pl
from jax.experimental.pallas import tpu as pltpu
from jax.experimental.pallas import tpu_sc as plsc

NC, NS, L = 2, 16, 16
NW = NC * NS
N_SRC = 4096
N_TGT = 16384
SRC_PER_TILE = N_SRC // NW
S = 4
G = 8
N_GROUPS = N_TGT // (G * L)
BIG = 3.0e38


def _round_bf16(x):
    u = lax.bitcast_convert_type(x, jnp.int32)
    r = (u + 0x7FFF + ((u >> 16) & 1)) & jnp.int32(-65536)
    return lax.bitcast_convert_type(r, jnp.float32)


def _sc_body(tgt_hbm, src_hbm, par_hbm, out_hbm,
             tgt_v, q_v, src_v, a_v, sq_v, p_v, res_v):
    c = lax.axis_index("c")
    s_ = lax.axis_index("s")
    wid = s_ * NC + c
    base = wid * SRC_PER_TILE

    pltpu.sync_copy(par_hbm, p_v)
    pltpu.sync_copy(tgt_hbm, tgt_v)
    for d in range(3):
        pltpu.sync_copy(src_hbm.at[pl.ds(d * N_SRC + base, SRC_PER_TILE)],
                        src_v.at[pl.ds(d * SRC_PER_TILE, SRC_PER_TILE)])

    pv = p_v[...]
    ev = jnp.exp(pv)
    sc = ev[0]
    t0 = pv[1]
    t1 = pv[2]
    t2 = pv[3]

    def src_prep(k, carry):
        px = src_v[pl.ds(k * L, L)] * sc + t0
        py = src_v[pl.ds(SRC_PER_TILE + k * L, L)] * sc + t1
        pz = src_v[pl.ds(2 * SRC_PER_TILE + k * L, L)] * sc + t2
        a_v[pl.ds(k * L, L)] = _round_bf16(px) * (-2.0)
        a_v[pl.ds(SRC_PER_TILE + k * L, L)] = _round_bf16(py) * (-2.0)
        a_v[pl.ds(2 * SRC_PER_TILE + k * L, L)] = _round_bf16(pz) * (-2.0)
        sq_v[pl.ds(k * L, L)] = px * px + py * py + pz * pz
        return carry

    lax.fori_loop(0, SRC_PER_TILE // L, src_prep, 0)

    def q_prep(j, carry):
        tx = tgt_v[pl.ds(j * L, L)]
        ty = tgt_v[pl.ds(N_TGT + j * L, L)]
        tz = tgt_v[pl.ds(2 * N_TGT + j * L, L)]
        q_v[pl.ds(j * L, L)] = tx * tx + ty * ty + tz * tz
        tgt_v[pl.ds(j * L, L)] = _round_bf16(tx)
        tgt_v[pl.ds(N_TGT + j * L, L)] = _round_bf16(ty)
        tgt_v[pl.ds(2 * N_TGT + j * L, L)] = _round_bf16(tz)
        return carry

    lax.fori_loop(0, N_TGT // L, q_prep, 0)

    def src_chunk(k, total):
        axv = a_v[pl.ds(k * L, L)]
        ayv = a_v[pl.ds(SRC_PER_TILE + k * L, L)]
        azv = a_v[pl.ds(2 * SRC_PER_TILE + k * L, L)]
        sqv = sq_v[pl.ds(k * L, L)]

        for sub in range(L // S):
            ax = [axv[sub * S + i] for i in range(S)]
            ay = [ayv[sub * S + i] for i in range(S)]
            az = [azv[sub * S + i] for i in range(S)]

            def group(g, maccs):
                off = g * (G * L)
                new = list(maccs)
                for cd in range(G):
                    o = off + cd * L
                    tx = tgt_v[pl.ds(o, L)]
                    ty = tgt_v[pl.ds(N_TGT + o, L)]
                    tz = tgt_v[pl.ds(2 * N_TGT + o, L)]
                    q = q_v[pl.ds(o, L)]
                    for i in range(S):
                        e = q + az[i] * tz
                        e = e + ay[i] * ty
                        e = e + ax[i] * tx
                        new[i] = jnp.minimum(new[i], e)
                return tuple(new)

            init = tuple(jnp.full((L,), BIG, jnp.float32) for _ in range(S))
            maccs = lax.fori_loop(0, N_GROUPS, group, init)
            for i in range(S):
                total = total + (sqv[sub * S + i] + jnp.min(maccs[i]))
        return total

    total = lax.fori_loop(0, SRC_PER_TILE // L, src_chunk, jnp.float32(0.0))

    res_v[...] = jnp.zeros((L,), jnp.float32) + total
    pltpu.sync_copy(res_v, out_hbm.at[wid])


_sc_nn = pl.kernel(
    _sc_body,
    out_type=jax.ShapeDtypeStruct((NW, L), jnp.float32),
    mesh=plsc.VectorSubcoreMesh(core_axis_name="c", subcore_axis_name="s",
                                num_cores=NC, num_subcores=NS),
    compiler_params=pltpu.CompilerParams(needs_layout_passes=False),
    scratch_types=[
        pltpu.VMEM((3 * N_TGT,), jnp.float32),
        pltpu.VMEM((N_TGT,), jnp.float32),
        pltpu.VMEM((3 * SRC_PER_TILE,), jnp.float32),
        pltpu.VMEM((3 * SRC_PER_TILE,), jnp.float32),
        pltpu.VMEM((SRC_PER_TILE,), jnp.float32),
        pltpu.VMEM((L,), jnp.float32),
        pltpu.VMEM((L,), jnp.float32),
    ],
)


def _finish_body(part_ref, scale_ref, out_ref):
    tot = jnp.sum(part_ref[:, 0:1], keepdims=True)
    out_ref[...] = tot / N_SRC + 0.1 * jnp.maximum(-scale_ref[...], 0.0)


def _finish(partials, scale2d):
    return pl.pallas_call(
        _finish_body,
        out_shape=jax.ShapeDtypeStruct((1, 1), jnp.float32),
    )(partials, scale2d)


def kernel(source_points, target_points, scale, translation):
    src_flat = source_points.T.reshape(-1)
    tgt_flat = target_points.T.reshape(-1)
    params = jnp.concatenate(
        [scale, translation, jnp.zeros((12,), jnp.float32)])
    partials = _sc_nn(tgt_flat, src_flat, params)
    loss = _finish(partials, scale.reshape(1, 1))
    return loss.reshape(1)
